# baseline (speedup 1.0000x reference)
import functools

import jax
import jax.numpy as jnp
from jax import lax
from jax.experimental import pallas as pl
from jax.experimental.pallas import tpu as pltpu

N_ENT = 50000
N_REL = 500
D = 128
QS = D // 4
EPS = 1e-5
BN = 1.0 / (1.0 + EPS) ** 0.5
B = 256


def _hamilton(w):
    r, i, j, k = jnp.split(w, 4, axis=1)
    r2 = jnp.concatenate([r, -i, -j, -k], axis=0)
    i2 = jnp.concatenate([i, r, -k, j], axis=0)
    j2 = jnp.concatenate([j, k, r, -i], axis=0)
    k2 = jnp.concatenate([k, -j, i, r], axis=0)
    return jnp.concatenate([r2, i2, j2, k2], axis=1)


_SC_BLK = 2048


def _scorer_body(h_ref, r_ref, x_ref, o_ref, hr_ref):
    @pl.when(pl.program_id(0) == 0)
    def _():
        q = h_ref[...]
        p = r_ref[...]
        pr, pi, pj, pk = (p[:, 0:QS], p[:, QS:2 * QS],
                          p[:, 2 * QS:3 * QS], p[:, 3 * QS:4 * QS])
        inv = lax.rsqrt(pr * pr + pi * pi + pj * pj + pk * pk)
        npr, npi, npj, npk = pr * inv, pi * inv, pj * inv, pk * inv
        qr, qi, qj, qk = (q[:, 0:QS], q[:, QS:2 * QS],
                          q[:, 2 * QS:3 * QS], q[:, 3 * QS:4 * QS])
        hr_r = qr * npr - qi * npi - qj * npj - qk * npk
        hr_i = qi * npr + qr * npi - qk * npj + qj * npk
        hr_j = qj * npr + qk * npi + qr * npj - qi * npk
        hr_k = qk * npr - qj * npi + qi * npj + qr * npk
        hr_ref[...] = jnp.concatenate([hr_r, hr_i, hr_j, hr_k], axis=1) * BN

    x = x_ref[...]
    logits = lax.dot_general(hr_ref[...], x, (((1,), (1,)), ((), ())),
                             preferred_element_type=jnp.float32)
    o_ref[...] = jax.nn.sigmoid(logits)


def _scorer(h, r, x):
    n = x.shape[0]
    grid = pl.cdiv(n, _SC_BLK)
    return pl.pallas_call(
        _scorer_body,
        grid=(grid,),
        in_specs=[
            pl.BlockSpec((B, D), lambda i: (0, 0)),
            pl.BlockSpec((B, D), lambda i: (0, 0)),
            pl.BlockSpec((_SC_BLK, D), lambda i: (i, 0)),
        ],
        out_specs=pl.BlockSpec((B, _SC_BLK), lambda i: (0, i)),
        out_shape=jax.ShapeDtypeStruct((B, n), jnp.float32),
        scratch_shapes=[pltpu.VMEM((B, D), jnp.float32)],
    )(h, r, x)


_MM_BLK = 4096


def _mm_body(x_ref, h_ref, o_ref):
    o_ref[...] = jnp.dot(x_ref[...], h_ref[...],
                         preferred_element_type=jnp.float32)


def _matmul(x, h):
    n = x.shape[0]
    grid = pl.cdiv(n, _MM_BLK)
    return pl.pallas_call(
        _mm_body,
        grid=(grid,),
        in_specs=[
            pl.BlockSpec((_MM_BLK, D), lambda i: (i, 0)),
            pl.BlockSpec((D, D), lambda i: (0, 0)),
        ],
        out_specs=pl.BlockSpec((_MM_BLK, D), lambda i: (i, 0)),
        out_shape=jax.ShapeDtypeStruct((n, D), jnp.float32),
    )(x, h)


_EW_BLK = 4096


def _combine_body(a_ref, b_ref, o_ref):
    o_ref[...] = jnp.tanh(a_ref[...] * BN) + jnp.tanh(b_ref[...] * BN)


def _combine(a, b):
    n = a.shape[0]
    return pl.pallas_call(
        _combine_body,
        grid=(pl.cdiv(n, _EW_BLK),),
        in_specs=[pl.BlockSpec((_EW_BLK, D), lambda i: (i, 0)),
                  pl.BlockSpec((_EW_BLK, D), lambda i: (i, 0))],
        out_specs=pl.BlockSpec((_EW_BLK, D), lambda i: (i, 0)),
        out_shape=jax.ShapeDtypeStruct((n, D), jnp.float32),
    )(a, b)


def _tanh_bn_body(a_ref, o_ref):
    o_ref[...] = jnp.tanh(a_ref[...] * BN)


def _tanh_bn(a):
    n = a.shape[0]
    return pl.pallas_call(
        _tanh_bn_body,
        grid=(pl.cdiv(n, _EW_BLK),),
        in_specs=[pl.BlockSpec((_EW_BLK, D), lambda i: (i, 0))],
        out_specs=pl.BlockSpec((_EW_BLK, D), lambda i: (i, 0)),
        out_shape=jax.ShapeDtypeStruct((n, D), jnp.float32),
    )(a)


def _spmm(rows, cols, vals, dense, n_out):
    gathered = jnp.take(dense, cols, axis=0) * vals[:, None]
    return jax.ops.segment_sum(gathered, rows, num_segments=n_out)


def kernel(e1_idx, r_idx, lst_indexes1, lst_indexes2, emb, w_gcn1, w_gcn2,
           adj_rows, adj_cols, adj_vals, adjr_rows, adjr_cols, adjr_vals):
    X = emb[:N_ENT]
    R = emb[N_ENT:]

    scores = [_scorer(jnp.take(X, e1_idx, axis=0),
                      jnp.take(R, r_idx, axis=0), X)]
    for l in range(2):
        XR = jnp.concatenate([X, R], axis=0)
        sup_r = _matmul(XR, _hamilton(w_gcn2[l]))
        sup_e = _matmul(X, _hamilton(w_gcn1[l]))
        agg_r = _spmm(adjr_rows, adjr_cols, adjr_vals, sup_r, N_ENT + N_REL)
        agg_e = _spmm(adj_rows, adj_cols, adj_vals, sup_e, N_ENT)
        X = _combine(agg_e, agg_r[:N_ENT])
        R = _tanh_bn(agg_r[N_ENT:])
        scores.append(_scorer(jnp.take(X, e1_idx, axis=0),
                              jnp.take(R, r_idx, axis=0), X))
    return tuple(scores)


# speedup vs baseline: 4.2692x; 4.2692x over previous
import functools

import jax
import jax.numpy as jnp
from jax import lax
from jax.experimental import pallas as pl
from jax.experimental.pallas import tpu as pltpu
from jax.experimental.pallas import tpu_sc as plsc

N_ENT = 50000
N_REL = 500
D = 128
QS = D // 4
EPS = 1e-5
BN = 1.0 / (1.0 + EPS) ** 0.5
B = 256


def _hamilton(w):
    r, i, j, k = jnp.split(w, 4, axis=1)
    r2 = jnp.concatenate([r, -i, -j, -k], axis=0)
    i2 = jnp.concatenate([i, r, -k, j], axis=0)
    j2 = jnp.concatenate([j, k, r, -i], axis=0)
    k2 = jnp.concatenate([k, -j, i, r], axis=0)
    return jnp.concatenate([r2, i2, j2, k2], axis=1)


_SC_BLK = 2048


def _scorer_body(h_ref, r_ref, x_ref, o_ref, hr_ref):
    @pl.when(pl.program_id(0) == 0)
    def _():
        q = h_ref[...]
        p = r_ref[...]
        pr, pi, pj, pk = (p[:, 0:QS], p[:, QS:2 * QS],
                          p[:, 2 * QS:3 * QS], p[:, 3 * QS:4 * QS])
        inv = lax.rsqrt(pr * pr + pi * pi + pj * pj + pk * pk)
        npr, npi, npj, npk = pr * inv, pi * inv, pj * inv, pk * inv
        qr, qi, qj, qk = (q[:, 0:QS], q[:, QS:2 * QS],
                          q[:, 2 * QS:3 * QS], q[:, 3 * QS:4 * QS])
        hr_r = qr * npr - qi * npi - qj * npj - qk * npk
        hr_i = qi * npr + qr * npi - qk * npj + qj * npk
        hr_j = qj * npr + qk * npi + qr * npj - qi * npk
        hr_k = qk * npr - qj * npi + qi * npj + qr * npk
        hr_ref[...] = jnp.concatenate([hr_r, hr_i, hr_j, hr_k], axis=1) * BN

    x = x_ref[...]
    logits = lax.dot_general(hr_ref[...], x, (((1,), (1,)), ((), ())),
                             preferred_element_type=jnp.float32)
    o_ref[...] = jax.nn.sigmoid(logits)


def _scorer(h, r, x):
    n = x.shape[0]
    grid = pl.cdiv(n, _SC_BLK)
    return pl.pallas_call(
        _scorer_body,
        grid=(grid,),
        in_specs=[
            pl.BlockSpec((B, D), lambda i: (0, 0)),
            pl.BlockSpec((B, D), lambda i: (0, 0)),
            pl.BlockSpec((_SC_BLK, D), lambda i: (i, 0)),
        ],
        out_specs=pl.BlockSpec((B, _SC_BLK), lambda i: (0, i)),
        out_shape=jax.ShapeDtypeStruct((B, n), jnp.float32),
        scratch_shapes=[pltpu.VMEM((B, D), jnp.float32)],
    )(h, r, x)


_MM_BLK = 4096


def _mm_body(x_ref, h_ref, o_ref):
    o_ref[...] = jnp.dot(x_ref[...], h_ref[...],
                         preferred_element_type=jnp.float32)


def _matmul(x, h):
    n = x.shape[0]
    grid = pl.cdiv(n, _MM_BLK)
    return pl.pallas_call(
        _mm_body,
        grid=(grid,),
        in_specs=[
            pl.BlockSpec((_MM_BLK, D), lambda i: (i, 0)),
            pl.BlockSpec((D, D), lambda i: (0, 0)),
        ],
        out_specs=pl.BlockSpec((_MM_BLK, D), lambda i: (i, 0)),
        out_shape=jax.ShapeDtypeStruct((n, D), jnp.float32),
    )(x, h)


def _mm_blk_body(x_ref, h_ref, o_ref):
    y = jnp.dot(x_ref[...], h_ref[...], preferred_element_type=jnp.float32)
    for q in range(4):
        o_ref[q] = y[:, q * 32:(q + 1) * 32]


def _matmul_blk(x, h, npad):
    return pl.pallas_call(
        _mm_blk_body,
        grid=(pl.cdiv(npad, _MM_BLK),),
        in_specs=[pl.BlockSpec((_MM_BLK, D), lambda i: (i, 0)),
                  pl.BlockSpec((D, D), lambda i: (0, 0))],
        out_specs=pl.BlockSpec((4, _MM_BLK, 32), lambda i: (0, i, 0)),
        out_shape=jax.ShapeDtypeStruct((4, npad, 32), jnp.float32),
    )(x, h)


_EW_BLK = 4096


def _combine_blk_body(a_ref, b_ref, o_ref):
    a = jnp.concatenate([a_ref[0], a_ref[1], a_ref[2], a_ref[3]], axis=1)
    b = jnp.concatenate([b_ref[0], b_ref[1], b_ref[2], b_ref[3]], axis=1)
    o_ref[...] = jnp.tanh(a * BN) + jnp.tanh(b * BN)


def _combine_blk(a, b, n):
    return pl.pallas_call(
        _combine_blk_body,
        grid=(pl.cdiv(n, _EW_BLK),),
        in_specs=[pl.BlockSpec((4, _EW_BLK, 32), lambda i: (0, i, 0)),
                  pl.BlockSpec((4, _EW_BLK, 32), lambda i: (0, i, 0))],
        out_specs=pl.BlockSpec((_EW_BLK, D), lambda i: (i, 0)),
        out_shape=jax.ShapeDtypeStruct((n, D), jnp.float32),
    )(a, b)


def _tanh_bn_body(a_ref, o_ref):
    o_ref[...] = jnp.tanh(a_ref[...] * BN)


def _tanh_bn(a):
    n = a.shape[0]
    return pl.pallas_call(
        _tanh_bn_body,
        grid=(pl.cdiv(n, _EW_BLK),),
        in_specs=[pl.BlockSpec((_EW_BLK, D), lambda i: (i, 0))],
        out_specs=pl.BlockSpec((_EW_BLK, D), lambda i: (i, 0)),
        out_shape=jax.ShapeDtypeStruct((n, D), jnp.float32),
    )(a)


_KW = 4
_CHUNK = _KW * 128
_NSUB = 16


@functools.lru_cache(maxsize=None)
def _make_sc_spmm(n_out, nch):
    npad = ((n_out + 127) // 128) * 128
    rchunk = npad // _NSUB
    sh = nch // _NSUB
    nchunks = sh // _KW
    assert sh % _KW == 0 and nch % _NSUB == 0

    mesh = plsc.VectorSubcoreMesh(core_axis_name="c", subcore_axis_name="s")

    @functools.partial(
        pl.kernel,
        out_type=jax.ShapeDtypeStruct((4 * npad, 32), jnp.float32),
        mesh=mesh,
        compiler_params=pltpu.CompilerParams(use_tc_tiling_on_sc=False),
        scratch_types=[
            pltpu.VMEM_SHARED((npad, 32), jnp.float32),
            pltpu.VMEM((_KW, 128), jnp.int32),
            pltpu.VMEM((_KW, 128), jnp.int32),
            pltpu.VMEM((_KW, 128), jnp.float32),
            pltpu.VMEM((_CHUNK, 32), jnp.float32),
            pltpu.SemaphoreType.DMA,
            pltpu.SemaphoreType.DMA,
            pltpu.SemaphoreType.DMA,
        ],
    )
    def spmm(sup_hbm, rows_hbm, cols_hbm, vals_hbm, out_hbm,
             acc, colw, roww, valw, gbuf, esem, gsem, ssem):
        c = lax.axis_index("c")
        s = lax.axis_index("s")
        r0 = s * rchunk
        w0 = s * sh

        zv = jnp.zeros((16,), jnp.float32)

        for p in range(2):
            blk = c + 2 * p
            off = blk * npad

            def zb(i, _):
                gbuf[i, 0:16] = zv
                gbuf[i, 16:32] = zv
                return 0
            lax.fori_loop(0, _CHUNK, zb, 0)
            for q0 in range(0, rchunk, _CHUNK):
                ln = min(_CHUNK, rchunk - q0)
                pltpu.sync_copy(gbuf.at[pl.ds(0, ln)],
                                acc.at[pl.ds(r0 + q0, ln)])
            plsc.subcore_barrier()

            offv = jnp.full((16,), off, jnp.int32)

            def chunk_body(t, _):
                r = w0 + t * _KW
                cp1 = pltpu.async_copy(cols_hbm.at[pl.ds(r, _KW)], colw, esem)
                cp2 = pltpu.async_copy(rows_hbm.at[pl.ds(r, _KW)], roww, esem)
                cp3 = pltpu.async_copy(vals_hbm.at[pl.ds(r, _KW)], valw, esem)
                cp1.wait()
                cp2.wait()
                cp3.wait()
                for k in range(_KW):
                    for g in range(8):
                        colw[k, pl.ds(g * 16, 16)] = (
                            colw[k, pl.ds(g * 16, 16)] + offv)
                gcps = [pltpu.async_copy(sup_hbm.at[colw.at[k]],
                                         gbuf.at[pl.ds(k * 128, 128)], gsem)
                        for k in range(_KW)]
                for cp in gcps:
                    cp.wait()
                for k in range(_KW):
                    def sgrp(g, _, k=k):
                        base = k * 128 + g * 16
                        vv = valw[k, pl.ds(g * 16, 16)]
                        for j in range(16):
                            vs = jnp.full((16,), vv[j], jnp.float32)
                            gbuf[base + j, 0:16] = gbuf[base + j, 0:16] * vs
                            gbuf[base + j, 16:32] = gbuf[base + j, 16:32] * vs
                        return 0
                    lax.fori_loop(0, 8, sgrp, 0)
                scps = [pltpu.async_copy(gbuf.at[pl.ds(k * 128, 128)],
                                         acc.at[roww.at[k]], ssem, add=True)
                        for k in range(_KW)]
                for cp in scps:
                    cp.wait()
                return 0

            lax.fori_loop(0, nchunks, chunk_body, 0)
            plsc.subcore_barrier()

            for q0 in range(0, rchunk, _CHUNK):
                ln = min(_CHUNK, rchunk - q0)
                pltpu.sync_copy(acc.at[pl.ds(r0 + q0, ln)],
                                out_hbm.at[pl.ds(off + r0 + q0, ln)])

    return spmm, npad


def _pad_edges(rows, cols, vals, n_out, nch):
    nnz = rows.shape[0]
    pad = nch * 128 - nnz
    fill = jnp.arange(pad, dtype=jnp.int32) % n_out
    rows = jnp.concatenate([rows, fill]).reshape(nch, 128)
    cols = jnp.concatenate([cols, fill]).reshape(nch, 128)
    vals = jnp.concatenate([vals, jnp.zeros((pad,), vals.dtype)]
                           ).reshape(nch, 128)
    return rows, cols, vals


def _spmm_sc(rows, cols, vals, sup_blk_flat, n_out, nch):
    fn, npad = _make_sc_spmm(n_out, nch)
    rows2, cols2, vals2 = _pad_edges(rows, cols, vals, n_out, nch)
    out = fn(sup_blk_flat, rows2, cols2, vals2)
    return out.reshape(4, npad, 32)


_NCH_E = 4736
_NCH_R = 5120
_NPAD_E = ((N_ENT + 127) // 128) * 128
_NPAD_R = ((N_ENT + N_REL + 127) // 128) * 128


def kernel(e1_idx, r_idx, lst_indexes1, lst_indexes2, emb, w_gcn1, w_gcn2,
           adj_rows, adj_cols, adj_vals, adjr_rows, adjr_cols, adjr_vals):
    X = emb[:N_ENT]
    R = emb[N_ENT:]

    scores = [_scorer(jnp.take(X, e1_idx, axis=0),
                      jnp.take(R, r_idx, axis=0), X)]
    for l in range(2):
        XR = jnp.concatenate([X, R], axis=0)
        sup_r = _matmul_blk(XR, _hamilton(w_gcn2[l]), _NPAD_R)
        sup_e = _matmul_blk(X, _hamilton(w_gcn1[l]), _NPAD_E)
        agg_r = _spmm_sc(adjr_rows, adjr_cols, adjr_vals,
                         sup_r.reshape(-1, 32), N_ENT + N_REL, _NCH_R)
        agg_e = _spmm_sc(adj_rows, adj_cols, adj_vals,
                         sup_e.reshape(-1, 32), N_ENT, _NCH_E)
        X = _combine_blk(agg_e, agg_r, N_ENT)
        rblk = jnp.transpose(agg_r[:, N_ENT:N_ENT + N_REL, :], (1, 0, 2))
        R = _tanh_bn(rblk.reshape(N_REL, D))
        scores.append(_scorer(jnp.take(X, e1_idx, axis=0),
                              jnp.take(R, r_idx, axis=0), X))
    return tuple(scores)


# speedup vs baseline: 4.7439x; 1.1112x over previous
import functools

import jax
import jax.numpy as jnp
from jax import lax
from jax.experimental import pallas as pl
from jax.experimental.pallas import tpu as pltpu
from jax.experimental.pallas import tpu_sc as plsc

N_ENT = 50000
N_REL = 500
D = 128
QS = D // 4
EPS = 1e-5
BN = 1.0 / (1.0 + EPS) ** 0.5
B = 256


def _hamilton(w):
    r, i, j, k = jnp.split(w, 4, axis=1)
    r2 = jnp.concatenate([r, -i, -j, -k], axis=0)
    i2 = jnp.concatenate([i, r, -k, j], axis=0)
    j2 = jnp.concatenate([j, k, r, -i], axis=0)
    k2 = jnp.concatenate([k, -j, i, r], axis=0)
    return jnp.concatenate([r2, i2, j2, k2], axis=1)


_SC_BLK = 2048


def _scorer_body(h_ref, r_ref, x_ref, o_ref, hr_ref):
    @pl.when(pl.program_id(0) == 0)
    def _():
        q = h_ref[...]
        p = r_ref[...]
        pr, pi, pj, pk = (p[:, 0:QS], p[:, QS:2 * QS],
                          p[:, 2 * QS:3 * QS], p[:, 3 * QS:4 * QS])
        inv = lax.rsqrt(pr * pr + pi * pi + pj * pj + pk * pk)
        npr, npi, npj, npk = pr * inv, pi * inv, pj * inv, pk * inv
        qr, qi, qj, qk = (q[:, 0:QS], q[:, QS:2 * QS],
                          q[:, 2 * QS:3 * QS], q[:, 3 * QS:4 * QS])
        hr_r = qr * npr - qi * npi - qj * npj - qk * npk
        hr_i = qi * npr + qr * npi - qk * npj + qj * npk
        hr_j = qj * npr + qk * npi + qr * npj - qi * npk
        hr_k = qk * npr - qj * npi + qi * npj + qr * npk
        hr_ref[...] = jnp.concatenate([hr_r, hr_i, hr_j, hr_k], axis=1) * BN

    x = x_ref[...]
    logits = lax.dot_general(hr_ref[...], x, (((1,), (1,)), ((), ())),
                             preferred_element_type=jnp.float32)
    o_ref[...] = jax.nn.sigmoid(logits)


def _scorer(h, r, x):
    n = x.shape[0]
    grid = pl.cdiv(n, _SC_BLK)
    return pl.pallas_call(
        _scorer_body,
        grid=(grid,),
        in_specs=[
            pl.BlockSpec((B, D), lambda i: (0, 0)),
            pl.BlockSpec((B, D), lambda i: (0, 0)),
            pl.BlockSpec((_SC_BLK, D), lambda i: (i, 0)),
        ],
        out_specs=pl.BlockSpec((B, _SC_BLK), lambda i: (0, i)),
        out_shape=jax.ShapeDtypeStruct((B, n), jnp.float32),
        scratch_shapes=[pltpu.VMEM((B, D), jnp.float32)],
    )(h, r, x)


_MM_BLK = 4096


def _mm_body(x_ref, h_ref, o_ref):
    o_ref[...] = jnp.dot(x_ref[...], h_ref[...],
                         preferred_element_type=jnp.float32)


def _matmul(x, h):
    n = x.shape[0]
    grid = pl.cdiv(n, _MM_BLK)
    return pl.pallas_call(
        _mm_body,
        grid=(grid,),
        in_specs=[
            pl.BlockSpec((_MM_BLK, D), lambda i: (i, 0)),
            pl.BlockSpec((D, D), lambda i: (0, 0)),
        ],
        out_specs=pl.BlockSpec((_MM_BLK, D), lambda i: (i, 0)),
        out_shape=jax.ShapeDtypeStruct((n, D), jnp.float32),
    )(x, h)


def _mm_blk_body(x_ref, h_ref, o_ref):
    y = jnp.dot(x_ref[...], h_ref[...], preferred_element_type=jnp.float32)
    for q in range(4):
        o_ref[q] = y[:, q * 32:(q + 1) * 32]


def _matmul_blk(x, h, npad):
    return pl.pallas_call(
        _mm_blk_body,
        grid=(pl.cdiv(npad, _MM_BLK),),
        in_specs=[pl.BlockSpec((_MM_BLK, D), lambda i: (i, 0)),
                  pl.BlockSpec((D, D), lambda i: (0, 0))],
        out_specs=pl.BlockSpec((4, _MM_BLK, 32), lambda i: (0, i, 0)),
        out_shape=jax.ShapeDtypeStruct((4, npad, 32), jnp.float32),
    )(x, h)


_EW_BLK = 4096


def _combine_blk_body(a_ref, b_ref, o_ref):
    a = jnp.concatenate([a_ref[0], a_ref[1], a_ref[2], a_ref[3]], axis=1)
    b = jnp.concatenate([b_ref[0], b_ref[1], b_ref[2], b_ref[3]], axis=1)
    o_ref[...] = jnp.tanh(a * BN) + jnp.tanh(b * BN)


def _combine_blk(a, b, n):
    return pl.pallas_call(
        _combine_blk_body,
        grid=(pl.cdiv(n, _EW_BLK),),
        in_specs=[pl.BlockSpec((4, _EW_BLK, 32), lambda i: (0, i, 0)),
                  pl.BlockSpec((4, _EW_BLK, 32), lambda i: (0, i, 0))],
        out_specs=pl.BlockSpec((_EW_BLK, D), lambda i: (i, 0)),
        out_shape=jax.ShapeDtypeStruct((n, D), jnp.float32),
    )(a, b)


def _tanh_bn_body(a_ref, o_ref):
    o_ref[...] = jnp.tanh(a_ref[...] * BN)


def _tanh_bn(a):
    n = a.shape[0]
    return pl.pallas_call(
        _tanh_bn_body,
        grid=(pl.cdiv(n, _EW_BLK),),
        in_specs=[pl.BlockSpec((_EW_BLK, D), lambda i: (i, 0))],
        out_specs=pl.BlockSpec((_EW_BLK, D), lambda i: (i, 0)),
        out_shape=jax.ShapeDtypeStruct((n, D), jnp.float32),
    )(a)


_KW = 2
_CHUNK = _KW * 128
_NSUB = 16


@functools.lru_cache(maxsize=None)
def _make_sc_spmm(n_out, nch):
    npad = ((n_out + 127) // 128) * 128
    rchunk = npad // _NSUB
    sh = nch // _NSUB
    nchunks = sh // _KW
    assert sh % _KW == 0 and nch % _NSUB == 0

    mesh = plsc.VectorSubcoreMesh(core_axis_name="c", subcore_axis_name="s")

    assert nchunks % 2 == 0 and nchunks >= 4

    @functools.partial(
        pl.kernel,
        out_type=jax.ShapeDtypeStruct((4 * npad, 32), jnp.float32),
        mesh=mesh,
        compiler_params=pltpu.CompilerParams(use_tc_tiling_on_sc=False),
        scratch_types=[
            pltpu.VMEM_SHARED((npad, 32), jnp.float32),
            [pltpu.VMEM((_KW, 128), jnp.int32)] * 2,
            [pltpu.VMEM((_KW, 128), jnp.int32)] * 2,
            [pltpu.VMEM((_KW, 128), jnp.float32)] * 2,
            [pltpu.VMEM((_KW, 128), jnp.int32)] * 2,
            [pltpu.VMEM((_CHUNK, 32), jnp.float32)] * 2,
            pltpu.SemaphoreType.DMA,
            pltpu.SemaphoreType.DMA,
            pltpu.SemaphoreType.DMA,
        ],
    )
    def spmm(sup_hbm, rows_hbm, cols_hbm, vals_hbm, out_hbm,
             acc, colw, roww, valw, rowsc, gbuf, esem, gsem, ssem):
        c = lax.axis_index("c")
        s = lax.axis_index("s")
        r0 = s * rchunk
        w0 = s * sh

        zv = jnp.zeros((16,), jnp.float32)

        def fire_edges(t, b):
            r = w0 + t * _KW
            pltpu.async_copy(cols_hbm.at[pl.ds(r, _KW)], colw[b], esem)
            pltpu.async_copy(rows_hbm.at[pl.ds(r, _KW)], roww[b], esem)
            pltpu.async_copy(vals_hbm.at[pl.ds(r, _KW)], valw[b], esem)

        def wait_edges(b):
            pltpu.make_async_copy(cols_hbm.at[pl.ds(0, _KW)], colw[b],
                                  esem).wait()
            pltpu.make_async_copy(rows_hbm.at[pl.ds(0, _KW)], roww[b],
                                  esem).wait()
            pltpu.make_async_copy(vals_hbm.at[pl.ds(0, _KW)], valw[b],
                                  esem).wait()

        def apply_off(b, offv):
            for k in range(_KW):
                for g in range(8):
                    colw[b][k, pl.ds(g * 16, 16)] = (
                        colw[b][k, pl.ds(g * 16, 16)] + offv)

        def fire_gathers(b):
            for k in range(_KW):
                pltpu.async_copy(sup_hbm.at[colw[b].at[k]],
                                 gbuf[b].at[pl.ds(k * 128, 128)], gsem)

        def wait_gathers(b):
            for k in range(_KW):
                pltpu.make_async_copy(sup_hbm.at[pl.ds(0, 128)],
                                      gbuf[b].at[pl.ds(k * 128, 128)],
                                      gsem).wait()

        def scale(b):
            for k in range(_KW):
                def sgrp(g, _, k=k):
                    base = k * 128 + g * 16
                    vv = valw[b][k, pl.ds(g * 16, 16)]
                    for j in range(16):
                        vs = jnp.full((16,), vv[j], jnp.float32)
                        gb = gbuf[b]
                        gb[base + j, 0:16] = gb[base + j, 0:16] * vs
                        gb[base + j, 16:32] = gb[base + j, 16:32] * vs
                    return 0
                lax.fori_loop(0, 8, sgrp, 0)

        def copy_rows(b):
            for k in range(_KW):
                for g in range(8):
                    rowsc[b][k, pl.ds(g * 16, 16)] = (
                        roww[b][k, pl.ds(g * 16, 16)])

        def fire_scatter(b):
            for k in range(_KW):
                pltpu.async_copy(gbuf[b].at[pl.ds(k * 128, 128)],
                                 acc.at[rowsc[b].at[k]], ssem, add=True)

        def wait_scatter(b):
            for k in range(_KW):
                pltpu.make_async_copy(sup_hbm.at[pl.ds(0, 128)],
                                      gbuf[b].at[pl.ds(k * 128, 128)],
                                      ssem).wait()

        def chunk_step(t, b, offv, guard):
            wait_edges(b)
            apply_off(b, offv)
            if guard:
                wait_scatter(b)
            fire_gathers(b)
            nxt = lax.rem(t + 1, nchunks)
            fire_edges(nxt, 1 - b)
            wait_gathers(b)
            scale(b)
            copy_rows(b)
            fire_scatter(b)

        fire_edges(0, 0)

        for p in range(2):
            blk = c + 2 * p
            off = blk * npad
            offv = jnp.full((16,), off, jnp.int32)

            def zb(i, _):
                gbuf[0][i, 0:16] = zv
                gbuf[0][i, 16:32] = zv
                return 0
            lax.fori_loop(0, _CHUNK, zb, 0)
            for q0 in range(0, rchunk, _CHUNK):
                ln = min(_CHUNK, rchunk - q0)
                pltpu.sync_copy(gbuf[0].at[pl.ds(0, ln)],
                                acc.at[pl.ds(r0 + q0, ln)])
            plsc.subcore_barrier()

            chunk_step(0, 0, offv, guard=False)
            chunk_step(1, 1, offv, guard=False)

            def steady(t2, _):
                for b in range(2):
                    chunk_step(2 * t2 + b, b, offv, guard=True)
                return 0
            lax.fori_loop(1, nchunks // 2, steady, 0)

            wait_scatter(0)
            wait_scatter(1)
            plsc.subcore_barrier()

            for q0 in range(0, rchunk, _CHUNK):
                ln = min(_CHUNK, rchunk - q0)
                pltpu.sync_copy(acc.at[pl.ds(r0 + q0, ln)],
                                out_hbm.at[pl.ds(off + r0 + q0, ln)])

        wait_edges(0)

    return spmm, npad


def _pad_edges(rows, cols, vals, n_out, nch):
    nnz = rows.shape[0]
    pad = nch * 128 - nnz
    fill = jnp.arange(pad, dtype=jnp.int32) % n_out
    rows = jnp.concatenate([rows, fill]).reshape(nch, 128)
    cols = jnp.concatenate([cols, fill]).reshape(nch, 128)
    vals = jnp.concatenate([vals, jnp.zeros((pad,), vals.dtype)]
                           ).reshape(nch, 128)
    return rows, cols, vals


def _spmm_sc(rows, cols, vals, sup_blk_flat, n_out, nch):
    fn, npad = _make_sc_spmm(n_out, nch)
    rows2, cols2, vals2 = _pad_edges(rows, cols, vals, n_out, nch)
    out = fn(sup_blk_flat, rows2, cols2, vals2)
    return out.reshape(4, npad, 32)


_NCH_E = 4736
_NCH_R = 5120
_NPAD_E = ((N_ENT + 127) // 128) * 128
_NPAD_R = ((N_ENT + N_REL + 127) // 128) * 128


def kernel(e1_idx, r_idx, lst_indexes1, lst_indexes2, emb, w_gcn1, w_gcn2,
           adj_rows, adj_cols, adj_vals, adjr_rows, adjr_cols, adjr_vals):
    X = emb[:N_ENT]
    R = emb[N_ENT:]

    scores = [_scorer(jnp.take(X, e1_idx, axis=0),
                      jnp.take(R, r_idx, axis=0), X)]
    for l in range(2):
        XR = jnp.concatenate([X, R], axis=0)
        sup_r = _matmul_blk(XR, _hamilton(w_gcn2[l]), _NPAD_R)
        sup_e = _matmul_blk(X, _hamilton(w_gcn1[l]), _NPAD_E)
        agg_r = _spmm_sc(adjr_rows, adjr_cols, adjr_vals,
                         sup_r.reshape(-1, 32), N_ENT + N_REL, _NCH_R)
        agg_e = _spmm_sc(adj_rows, adj_cols, adj_vals,
                         sup_e.reshape(-1, 32), N_ENT, _NCH_E)
        X = _combine_blk(agg_e, agg_r, N_ENT)
        rblk = jnp.transpose(agg_r[:, N_ENT:N_ENT + N_REL, :], (1, 0, 2))
        R = _tanh_bn(rblk.reshape(N_REL, D))
        scores.append(_scorer(jnp.take(X, e1_idx, axis=0),
                              jnp.take(R, r_idx, axis=0), X))
    return tuple(scores)


# speedup vs baseline: 6.0550x; 1.2764x over previous
import functools

import jax
import jax.numpy as jnp
from jax import lax
from jax.experimental import pallas as pl
from jax.experimental.pallas import tpu as pltpu
from jax.experimental.pallas import tpu_sc as plsc

N_ENT = 50000
N_REL = 500
D = 128
QS = D // 4
EPS = 1e-5
BN = 1.0 / (1.0 + EPS) ** 0.5
B = 256


def _hamilton(w):
    r, i, j, k = jnp.split(w, 4, axis=1)
    r2 = jnp.concatenate([r, -i, -j, -k], axis=0)
    i2 = jnp.concatenate([i, r, -k, j], axis=0)
    j2 = jnp.concatenate([j, k, r, -i], axis=0)
    k2 = jnp.concatenate([k, -j, i, r], axis=0)
    return jnp.concatenate([r2, i2, j2, k2], axis=1)


_SC_BLK = 2048


def _scorer_body(h_ref, r_ref, x_ref, o_ref, hr_ref):
    @pl.when(pl.program_id(0) == 0)
    def _():
        q = h_ref[...]
        p = r_ref[...]
        pr, pi, pj, pk = (p[:, 0:QS], p[:, QS:2 * QS],
                          p[:, 2 * QS:3 * QS], p[:, 3 * QS:4 * QS])
        inv = lax.rsqrt(pr * pr + pi * pi + pj * pj + pk * pk)
        npr, npi, npj, npk = pr * inv, pi * inv, pj * inv, pk * inv
        qr, qi, qj, qk = (q[:, 0:QS], q[:, QS:2 * QS],
                          q[:, 2 * QS:3 * QS], q[:, 3 * QS:4 * QS])
        hr_r = qr * npr - qi * npi - qj * npj - qk * npk
        hr_i = qi * npr + qr * npi - qk * npj + qj * npk
        hr_j = qj * npr + qk * npi + qr * npj - qi * npk
        hr_k = qk * npr - qj * npi + qi * npj + qr * npk
        hr_ref[...] = jnp.concatenate([hr_r, hr_i, hr_j, hr_k], axis=1) * BN

    x = x_ref[...]
    logits = lax.dot_general(hr_ref[...], x, (((1,), (1,)), ((), ())),
                             preferred_element_type=jnp.float32)
    o_ref[...] = jax.nn.sigmoid(logits)


def _scorer(h, r, x):
    n = x.shape[0]
    grid = pl.cdiv(n, _SC_BLK)
    return pl.pallas_call(
        _scorer_body,
        grid=(grid,),
        in_specs=[
            pl.BlockSpec((B, D), lambda i: (0, 0)),
            pl.BlockSpec((B, D), lambda i: (0, 0)),
            pl.BlockSpec((_SC_BLK, D), lambda i: (i, 0)),
        ],
        out_specs=pl.BlockSpec((B, _SC_BLK), lambda i: (0, i)),
        out_shape=jax.ShapeDtypeStruct((B, n), jnp.float32),
        scratch_shapes=[pltpu.VMEM((B, D), jnp.float32)],
    )(h, r, x)


_MM_BLK = 4096


def _mm_body(x_ref, h_ref, o_ref):
    o_ref[...] = jnp.dot(x_ref[...], h_ref[...],
                         preferred_element_type=jnp.float32)


def _matmul(x, h):
    n = x.shape[0]
    grid = pl.cdiv(n, _MM_BLK)
    return pl.pallas_call(
        _mm_body,
        grid=(grid,),
        in_specs=[
            pl.BlockSpec((_MM_BLK, D), lambda i: (i, 0)),
            pl.BlockSpec((D, D), lambda i: (0, 0)),
        ],
        out_specs=pl.BlockSpec((_MM_BLK, D), lambda i: (i, 0)),
        out_shape=jax.ShapeDtypeStruct((n, D), jnp.float32),
    )(x, h)


def _mm_blk_body(x_ref, h_ref, o_ref):
    y = jnp.dot(x_ref[...], h_ref[...], preferred_element_type=jnp.float32)
    for q in range(4):
        o_ref[q] = y[:, q * 32:(q + 1) * 32]


def _matmul_blk(x, h, npad):
    return pl.pallas_call(
        _mm_blk_body,
        grid=(pl.cdiv(npad, _MM_BLK),),
        in_specs=[pl.BlockSpec((_MM_BLK, D), lambda i: (i, 0)),
                  pl.BlockSpec((D, D), lambda i: (0, 0))],
        out_specs=pl.BlockSpec((4, _MM_BLK, 32), lambda i: (0, i, 0)),
        out_shape=jax.ShapeDtypeStruct((4, npad, 32), jnp.float32),
    )(x, h)


_EW_BLK = 4096


def _combine_blk_body(a_ref, b_ref, o_ref):
    a = jnp.concatenate([a_ref[0], a_ref[1], a_ref[2], a_ref[3]], axis=1)
    b = jnp.concatenate([b_ref[0], b_ref[1], b_ref[2], b_ref[3]], axis=1)
    o_ref[...] = jnp.tanh(a * BN) + jnp.tanh(b * BN)


def _combine_blk(a, b, n):
    return pl.pallas_call(
        _combine_blk_body,
        grid=(pl.cdiv(n, _EW_BLK),),
        in_specs=[pl.BlockSpec((4, _EW_BLK, 32), lambda i: (0, i, 0)),
                  pl.BlockSpec((4, _EW_BLK, 32), lambda i: (0, i, 0))],
        out_specs=pl.BlockSpec((_EW_BLK, D), lambda i: (i, 0)),
        out_shape=jax.ShapeDtypeStruct((n, D), jnp.float32),
    )(a, b)


def _tanh_bn_body(a_ref, o_ref):
    o_ref[...] = jnp.tanh(a_ref[...] * BN)


def _tanh_bn(a):
    n = a.shape[0]
    return pl.pallas_call(
        _tanh_bn_body,
        grid=(pl.cdiv(n, _EW_BLK),),
        in_specs=[pl.BlockSpec((_EW_BLK, D), lambda i: (i, 0))],
        out_specs=pl.BlockSpec((_EW_BLK, D), lambda i: (i, 0)),
        out_shape=jax.ShapeDtypeStruct((n, D), jnp.float32),
    )(a)


_KW = 2
_CHUNK = _KW * 128
_NSUB = 16


@functools.lru_cache(maxsize=None)
def _make_sc_spmm(n_out, nch):
    npad = ((n_out + 127) // 128) * 128
    rchunk = npad // _NSUB
    sh = nch // _NSUB
    nchunks = sh // _KW
    assert sh % _KW == 0 and nch % _NSUB == 0

    mesh = plsc.VectorSubcoreMesh(core_axis_name="c", subcore_axis_name="s")

    assert nchunks % 4 == 0 and nchunks >= 8

    @functools.partial(
        pl.kernel,
        out_type=jax.ShapeDtypeStruct((4 * npad, 32), jnp.float32),
        mesh=mesh,
        compiler_params=pltpu.CompilerParams(use_tc_tiling_on_sc=False),
        scratch_types=[
            pltpu.VMEM_SHARED((npad, 32), jnp.float32),
            [pltpu.VMEM((_KW, 128), jnp.int32)] * 4,
            [pltpu.VMEM((_KW, 128), jnp.int32)] * 4,
            [pltpu.VMEM((_KW, 128), jnp.float32)] * 4,
            [pltpu.VMEM((_KW, 128), jnp.int32)] * 2,
            [pltpu.VMEM((_CHUNK, 32), jnp.float32)] * 2,
            pltpu.SemaphoreType.DMA,
            [pltpu.SemaphoreType.DMA] * 2,
            [pltpu.SemaphoreType.DMA] * 2,
        ],
    )
    def spmm(sup_hbm, rows_hbm, cols_hbm, vals_hbm, out_hbm,
             acc, colw, roww, valw, rowsc, gbuf, esem, gsem, ssem):
        c = lax.axis_index("c")
        s = lax.axis_index("s")
        r0 = s * rchunk
        w0 = s * sh

        zv = jnp.zeros((16,), jnp.float32)

        def fire_edges(t, b):
            r = w0 + t * _KW
            pltpu.async_copy(cols_hbm.at[pl.ds(r, _KW)], colw[b], esem)
            pltpu.async_copy(rows_hbm.at[pl.ds(r, _KW)], roww[b], esem)
            pltpu.async_copy(vals_hbm.at[pl.ds(r, _KW)], valw[b], esem)

        def wait_edges(b):
            pltpu.make_async_copy(cols_hbm.at[pl.ds(0, _KW)], colw[b],
                                  esem).wait()
            pltpu.make_async_copy(rows_hbm.at[pl.ds(0, _KW)], roww[b],
                                  esem).wait()
            pltpu.make_async_copy(vals_hbm.at[pl.ds(0, _KW)], valw[b],
                                  esem).wait()

        def apply_off(b, offv):
            for k in range(_KW):
                for g in range(8):
                    colw[b][k, pl.ds(g * 16, 16)] = (
                        colw[b][k, pl.ds(g * 16, 16)] + offv)

        def wait_gathers(bg):
            for k in range(_KW):
                pltpu.make_async_copy(sup_hbm.at[pl.ds(0, 128)],
                                      gbuf[bg].at[pl.ds(k * 128, 128)],
                                      gsem[bg]).wait()

        def scale(bg, eb):
            for k in range(_KW):
                def sgrp(g, _, k=k):
                    base = k * 128 + g * 16
                    vv = valw[eb][k, pl.ds(g * 16, 16)]
                    for j in range(16):
                        vs = jnp.full((16,), vv[j], jnp.float32)
                        gb = gbuf[bg]
                        gb[base + j, 0:16] = gb[base + j, 0:16] * vs
                        gb[base + j, 16:32] = gb[base + j, 16:32] * vs
                    return 0
                lax.fori_loop(0, 8, sgrp, 0)

        def copy_rows(bg, eb):
            for k in range(_KW):
                for g in range(8):
                    rowsc[bg][k, pl.ds(g * 16, 16)] = (
                        roww[eb][k, pl.ds(g * 16, 16)])

        def fire_scatter(bg):
            for k in range(_KW):
                pltpu.async_copy(gbuf[bg].at[pl.ds(k * 128, 128)],
                                 acc.at[rowsc[bg].at[k]], ssem[bg], add=True)

        def wait_scatter(bg):
            for k in range(_KW):
                pltpu.make_async_copy(sup_hbm.at[pl.ds(0, 128)],
                                      gbuf[bg].at[pl.ds(k * 128, 128)],
                                      ssem[bg]).wait()

        def front(t, eb, offv, guard):
            wait_edges(eb)
            apply_off(eb, offv)
            if guard:
                wait_scatter(eb % 2)
            fire_gathers2(t, eb)
            nxt = lax.rem(t + 1, nchunks)
            fire_edges(nxt, (eb + 1) % 4)

        def fire_gathers2(t, eb):
            for k in range(_KW):
                pltpu.async_copy(sup_hbm.at[colw[eb].at[k]],
                                 gbuf[eb % 2].at[pl.ds(k * 128, 128)],
                                 gsem[eb % 2])

        def back(eb):
            bg = eb % 2
            wait_gathers(bg)
            scale(bg, eb)
            copy_rows(bg, eb)
            fire_scatter(bg)

        fire_edges(0, 0)

        for p in range(2):
            blk = c + 2 * p
            off = blk * npad
            offv = jnp.full((16,), off, jnp.int32)

            def zb(i, _):
                gbuf[0][i, 0:16] = zv
                gbuf[0][i, 16:32] = zv
                return 0
            lax.fori_loop(0, _CHUNK, zb, 0)
            for q0 in range(0, rchunk, _CHUNK):
                ln = min(_CHUNK, rchunk - q0)
                pltpu.sync_copy(gbuf[0].at[pl.ds(0, ln)],
                                acc.at[pl.ds(r0 + q0, ln)])
            plsc.subcore_barrier()

            front(0, 0, offv, guard=False)
            front(1, 1, offv, guard=False)
            back(0)
            front(2, 2, offv, guard=True)
            back(1)
            front(3, 3, offv, guard=True)
            back(2)

            def steady(t4, _):
                for u in range(4):
                    front(4 * t4 + u, u, offv, guard=True)
                    back((u + 3) % 4)
                return 0
            lax.fori_loop(1, nchunks // 4, steady, 0)

            back((nchunks - 1) % 4)
            wait_scatter(0)
            wait_scatter(1)
            plsc.subcore_barrier()

            for q0 in range(0, rchunk, _CHUNK):
                ln = min(_CHUNK, rchunk - q0)
                pltpu.sync_copy(acc.at[pl.ds(r0 + q0, ln)],
                                out_hbm.at[pl.ds(off + r0 + q0, ln)])

        wait_edges(0)

    return spmm, npad


def _pad_edges(rows, cols, vals, n_out, nch):
    nnz = rows.shape[0]
    pad = nch * 128 - nnz
    fill = jnp.arange(pad, dtype=jnp.int32) % n_out
    rows = jnp.concatenate([rows, fill]).reshape(nch, 128)
    cols = jnp.concatenate([cols, fill]).reshape(nch, 128)
    vals = jnp.concatenate([vals, jnp.zeros((pad,), vals.dtype)]
                           ).reshape(nch, 128)
    return rows, cols, vals


def _spmm_sc(rows, cols, vals, sup_blk_flat, n_out, nch):
    fn, npad = _make_sc_spmm(n_out, nch)
    rows2, cols2, vals2 = _pad_edges(rows, cols, vals, n_out, nch)
    out = fn(sup_blk_flat, rows2, cols2, vals2)
    return out.reshape(4, npad, 32)


_NCH_E = 4736
_NCH_R = 5120
_NPAD_E = ((N_ENT + 127) // 128) * 128
_NPAD_R = ((N_ENT + N_REL + 127) // 128) * 128


def kernel(e1_idx, r_idx, lst_indexes1, lst_indexes2, emb, w_gcn1, w_gcn2,
           adj_rows, adj_cols, adj_vals, adjr_rows, adjr_cols, adjr_vals):
    X = emb[:N_ENT]
    R = emb[N_ENT:]

    scores = [_scorer(jnp.take(X, e1_idx, axis=0),
                      jnp.take(R, r_idx, axis=0), X)]
    for l in range(2):
        XR = jnp.concatenate([X, R], axis=0)
        sup_r = _matmul_blk(XR, _hamilton(w_gcn2[l]), _NPAD_R)
        sup_e = _matmul_blk(X, _hamilton(w_gcn1[l]), _NPAD_E)
        agg_r = _spmm_sc(adjr_rows, adjr_cols, adjr_vals,
                         sup_r.reshape(-1, 32), N_ENT + N_REL, _NCH_R)
        agg_e = _spmm_sc(adj_rows, adj_cols, adj_vals,
                         sup_e.reshape(-1, 32), N_ENT, _NCH_E)
        X = _combine_blk(agg_e, agg_r, N_ENT)
        rblk = jnp.transpose(agg_r[:, N_ENT:N_ENT + N_REL, :], (1, 0, 2))
        R = _tanh_bn(rblk.reshape(N_REL, D))
        scores.append(_scorer(jnp.take(X, e1_idx, axis=0),
                              jnp.take(R, r_idx, axis=0), X))
    return tuple(scores)
